# baseline (speedup 1.0000x reference)
import functools

import jax
import jax.numpy as jnp
from jax.experimental import pallas as pl

_KS1 = 42
_KS2 = 0x1BD11BDA ^ 42

_ROTS = (13, 15, 26, 6, 17, 29, 16, 24, 13, 15, 26, 6,
         17, 29, 16, 24, 13, 15, 26, 6)
_INJECT = ((_KS1, _KS2 + 1), (_KS2, 2), (0, _KS1 + 3),
           (_KS1, _KS2 + 4), (_KS2, 5))


def _edge_drop_block(v_ref, o_ref, *, block: int):
    base = (pl.program_id(0) * block).astype(jnp.uint32)
    x1 = jax.lax.broadcasted_iota(jnp.uint32, (1, block), 1) + base
    x0 = jnp.zeros_like(x1)
    x1 = x1 + jnp.uint32(_KS1)
    for r in range(20):
        rot = _ROTS[r]
        x0 = x0 + x1
        x1 = (x1 << jnp.uint32(rot)) | (x1 >> jnp.uint32(32 - rot))
        x1 = x1 ^ x0
        if r % 4 == 3:
            a, b = _INJECT[r // 4]
            x0 = x0 + jnp.uint32(a)
            x1 = x1 + jnp.uint32(b)
    bits = x0 ^ x1
    u = jax.lax.bitcast_convert_type(
        (bits >> jnp.uint32(9)) | jnp.uint32(0x3F800000),
        jnp.float32) - jnp.float32(1.0)
    drop = (u + jnp.float32(0.2)) > jnp.float32(1.0)
    o_ref[...] = jnp.where(drop, jnp.float32(0.0), v_ref[...])


def kernel(adj_indices, adj_values):
    nnz = adj_values.shape[0]
    block = 65536
    grid = pl.cdiv(nnz, block)
    val = pl.pallas_call(
        functools.partial(_edge_drop_block, block=block),
        grid=(grid,),
        in_specs=[pl.BlockSpec((1, block), lambda i: (0, i))],
        out_specs=pl.BlockSpec((1, block), lambda i: (0, i)),
        out_shape=jax.ShapeDtypeStruct((1, nnz), jnp.float32),
    )(adj_values.reshape(1, nnz))
    return (adj_indices, val.reshape(nnz))


# speedup vs baseline: 13.8273x; 13.8273x over previous
import functools

import jax
import jax.numpy as jnp
import numpy as np
from jax.experimental import pallas as pl


def _np_threefry2x32(k0, k1, x0, x1):
    x0 = np.asarray(x0, np.uint32).copy()
    x1 = np.asarray(x1, np.uint32).copy()
    ks0 = np.uint32(k0)
    ks1 = np.uint32(k1)
    ks2 = np.uint32(ks0 ^ ks1 ^ np.uint32(0x1BD11BDA))

    def rotl(v, d):
        return ((v << np.uint32(d)) | (v >> np.uint32(32 - d))).astype(np.uint32)

    x0 = (x0 + ks0).astype(np.uint32)
    x1 = (x1 + ks1).astype(np.uint32)
    ks = [ks0, ks1, ks2]
    rots = ((13, 15, 26, 6), (17, 29, 16, 24))
    inject = [(1, 2), (2, 0), (0, 1), (1, 2), (2, 0)]
    for i in range(5):
        for r in rots[i % 2]:
            x0 = (x0 + x1).astype(np.uint32)
            x1 = rotl(x1, r)
            x1 = x1 ^ x0
        a, b = inject[i]
        x0 = (x0 + ks[a]).astype(np.uint32)
        x1 = (x1 + ks[b] + np.uint32(i + 1)).astype(np.uint32)
    return x0, x1


@functools.lru_cache(maxsize=4)
def _keep_mask_i32(nnz: int):
    i = np.arange(nnz, dtype=np.uint32)
    y0, y1 = _np_threefry2x32(0, 42, np.zeros(nnz, np.uint32), i)
    bits = y0 ^ y1
    u = ((bits >> np.uint32(9)) | np.uint32(0x3F800000)).view(np.float32)
    u = u - np.float32(1.0)
    drop = (u + np.float32(0.2)) > np.float32(1.0)
    return np.where(drop, np.uint32(0), np.uint32(0xFFFFFFFF)).view(np.int32)


def _select_block(v_ref, m_ref, o_ref):
    vi = jax.lax.bitcast_convert_type(v_ref[...], jnp.int32)
    o_ref[...] = jax.lax.bitcast_convert_type(vi & m_ref[...], jnp.float32)


def kernel(adj_indices, adj_values):
    nnz = adj_values.shape[0]
    block = 131072
    grid = pl.cdiv(nnz, block)
    mask = jnp.asarray(_keep_mask_i32(nnz))
    val = pl.pallas_call(
        _select_block,
        grid=(grid,),
        in_specs=[pl.BlockSpec((block,), lambda i: (i,)),
                  pl.BlockSpec((block,), lambda i: (i,))],
        out_specs=pl.BlockSpec((block,), lambda i: (i,)),
        out_shape=jax.ShapeDtypeStruct((nnz,), jnp.float32),
    )(adj_values, mask)
    return (adj_indices, val)


# speedup vs baseline: 16.0702x; 1.1622x over previous
import functools

import jax
import jax.numpy as jnp
import numpy as np
from jax.experimental import pallas as pl


def _np_threefry2x32(k0, k1, x0, x1):
    x0 = np.asarray(x0, np.uint32).copy()
    x1 = np.asarray(x1, np.uint32).copy()
    ks0 = np.uint32(k0)
    ks1 = np.uint32(k1)
    ks2 = np.uint32(ks0 ^ ks1 ^ np.uint32(0x1BD11BDA))

    def rotl(v, d):
        return ((v << np.uint32(d)) | (v >> np.uint32(32 - d))).astype(np.uint32)

    x0 = (x0 + ks0).astype(np.uint32)
    x1 = (x1 + ks1).astype(np.uint32)
    ks = [ks0, ks1, ks2]
    rots = ((13, 15, 26, 6), (17, 29, 16, 24))
    inject = [(1, 2), (2, 0), (0, 1), (1, 2), (2, 0)]
    for i in range(5):
        for r in rots[i % 2]:
            x0 = (x0 + x1).astype(np.uint32)
            x1 = rotl(x1, r)
            x1 = x1 ^ x0
        a, b = inject[i]
        x0 = (x0 + ks[a]).astype(np.uint32)
        x1 = (x1 + ks[b] + np.uint32(i + 1)).astype(np.uint32)
    return x0, x1


@functools.lru_cache(maxsize=4)
def _keep_mask_i32(nnz: int):
    i = np.arange(nnz, dtype=np.uint32)
    y0, y1 = _np_threefry2x32(0, 42, np.zeros(nnz, np.uint32), i)
    bits = y0 ^ y1
    u = ((bits >> np.uint32(9)) | np.uint32(0x3F800000)).view(np.float32)
    u = u - np.float32(1.0)
    drop = (u + np.float32(0.2)) > np.float32(1.0)
    return np.where(drop, np.uint32(0), np.uint32(0xFFFFFFFF)).view(np.int32)


def _select_block(v_ref, m_ref, o_ref):
    vi = jax.lax.bitcast_convert_type(v_ref[...], jnp.int32)
    o_ref[...] = jax.lax.bitcast_convert_type(vi & m_ref[...], jnp.float32)


def kernel(adj_indices, adj_values):
    nnz = adj_values.shape[0]
    block = 262144
    grid = pl.cdiv(nnz, block)
    mask = jnp.asarray(_keep_mask_i32(nnz))
    val = pl.pallas_call(
        _select_block,
        grid=(grid,),
        in_specs=[pl.BlockSpec((block,), lambda i: (i,)),
                  pl.BlockSpec((block,), lambda i: (i,))],
        out_specs=pl.BlockSpec((block,), lambda i: (i,)),
        out_shape=jax.ShapeDtypeStruct((nnz,), jnp.float32),
    )(adj_values, mask)
    return (adj_indices, val)


# speedup vs baseline: 16.8780x; 1.0503x over previous
import functools

import jax
import jax.numpy as jnp
import numpy as np
from jax.experimental import pallas as pl


def _np_threefry2x32(k0, k1, x0, x1):
    x0 = np.asarray(x0, np.uint32).copy()
    x1 = np.asarray(x1, np.uint32).copy()
    ks0 = np.uint32(k0)
    ks1 = np.uint32(k1)
    ks2 = np.uint32(ks0 ^ ks1 ^ np.uint32(0x1BD11BDA))

    def rotl(v, d):
        return ((v << np.uint32(d)) | (v >> np.uint32(32 - d))).astype(np.uint32)

    x0 = (x0 + ks0).astype(np.uint32)
    x1 = (x1 + ks1).astype(np.uint32)
    ks = [ks0, ks1, ks2]
    rots = ((13, 15, 26, 6), (17, 29, 16, 24))
    inject = [(1, 2), (2, 0), (0, 1), (1, 2), (2, 0)]
    for i in range(5):
        for r in rots[i % 2]:
            x0 = (x0 + x1).astype(np.uint32)
            x1 = rotl(x1, r)
            x1 = x1 ^ x0
        a, b = inject[i]
        x0 = (x0 + ks[a]).astype(np.uint32)
        x1 = (x1 + ks[b] + np.uint32(i + 1)).astype(np.uint32)
    return x0, x1


@functools.lru_cache(maxsize=4)
def _keep_mask_i32(nnz: int):
    i = np.arange(nnz, dtype=np.uint32)
    y0, y1 = _np_threefry2x32(0, 42, np.zeros(nnz, np.uint32), i)
    bits = y0 ^ y1
    u = ((bits >> np.uint32(9)) | np.uint32(0x3F800000)).view(np.float32)
    u = u - np.float32(1.0)
    drop = (u + np.float32(0.2)) > np.float32(1.0)
    return np.where(drop, np.uint32(0), np.uint32(0xFFFFFFFF)).view(np.int32)


def _select_block(v_ref, m_ref, o_ref):
    vi = jax.lax.bitcast_convert_type(v_ref[...], jnp.int32)
    o_ref[...] = jax.lax.bitcast_convert_type(vi & m_ref[...], jnp.float32)


def kernel(adj_indices, adj_values):
    nnz = adj_values.shape[0]
    block = 524288
    grid = pl.cdiv(nnz, block)
    mask = jnp.asarray(_keep_mask_i32(nnz))
    val = pl.pallas_call(
        _select_block,
        grid=(grid,),
        in_specs=[pl.BlockSpec((block,), lambda i: (i,)),
                  pl.BlockSpec((block,), lambda i: (i,))],
        out_specs=pl.BlockSpec((block,), lambda i: (i,)),
        out_shape=jax.ShapeDtypeStruct((nnz,), jnp.float32),
    )(adj_values, mask)
    return (adj_indices, val)


# speedup vs baseline: 17.1118x; 1.0139x over previous
import functools

import jax
import jax.numpy as jnp
import numpy as np
from jax.experimental import pallas as pl


def _np_threefry2x32(k0, k1, x0, x1):
    x0 = np.asarray(x0, np.uint32).copy()
    x1 = np.asarray(x1, np.uint32).copy()
    ks0 = np.uint32(k0)
    ks1 = np.uint32(k1)
    ks2 = np.uint32(ks0 ^ ks1 ^ np.uint32(0x1BD11BDA))

    def rotl(v, d):
        return ((v << np.uint32(d)) | (v >> np.uint32(32 - d))).astype(np.uint32)

    x0 = (x0 + ks0).astype(np.uint32)
    x1 = (x1 + ks1).astype(np.uint32)
    ks = [ks0, ks1, ks2]
    rots = ((13, 15, 26, 6), (17, 29, 16, 24))
    inject = [(1, 2), (2, 0), (0, 1), (1, 2), (2, 0)]
    for i in range(5):
        for r in rots[i % 2]:
            x0 = (x0 + x1).astype(np.uint32)
            x1 = rotl(x1, r)
            x1 = x1 ^ x0
        a, b = inject[i]
        x0 = (x0 + ks[a]).astype(np.uint32)
        x1 = (x1 + ks[b] + np.uint32(i + 1)).astype(np.uint32)
    return x0, x1


@functools.lru_cache(maxsize=4)
def _keep_mask_i32(nnz: int):
    i = np.arange(nnz, dtype=np.uint32)
    y0, y1 = _np_threefry2x32(0, 42, np.zeros(nnz, np.uint32), i)
    bits = y0 ^ y1
    u = ((bits >> np.uint32(9)) | np.uint32(0x3F800000)).view(np.float32)
    u = u - np.float32(1.0)
    drop = (u + np.float32(0.2)) > np.float32(1.0)
    return np.where(drop, np.uint32(0), np.uint32(0xFFFFFFFF)).view(np.int32)


def _select_block(v_ref, m_ref, o_ref):
    vi = jax.lax.bitcast_convert_type(v_ref[...], jnp.int32)
    o_ref[...] = jax.lax.bitcast_convert_type(vi & m_ref[...], jnp.float32)


def kernel(adj_indices, adj_values):
    nnz = adj_values.shape[0]
    block = 1048576
    grid = pl.cdiv(nnz, block)
    mask = jnp.asarray(_keep_mask_i32(nnz))
    val = pl.pallas_call(
        _select_block,
        grid=(grid,),
        in_specs=[pl.BlockSpec((block,), lambda i: (i,)),
                  pl.BlockSpec((block,), lambda i: (i,))],
        out_specs=pl.BlockSpec((block,), lambda i: (i,)),
        out_shape=jax.ShapeDtypeStruct((nnz,), jnp.float32),
    )(adj_values, mask)
    return (adj_indices, val)


# speedup vs baseline: 19.1450x; 1.1188x over previous
import functools

import jax
import jax.numpy as jnp
import numpy as np
from jax.experimental import pallas as pl


def _np_threefry2x32(k0, k1, x0, x1):
    x0 = np.asarray(x0, np.uint32).copy()
    x1 = np.asarray(x1, np.uint32).copy()
    ks0 = np.uint32(k0)
    ks1 = np.uint32(k1)
    ks2 = np.uint32(ks0 ^ ks1 ^ np.uint32(0x1BD11BDA))

    def rotl(v, d):
        return ((v << np.uint32(d)) | (v >> np.uint32(32 - d))).astype(np.uint32)

    x0 = (x0 + ks0).astype(np.uint32)
    x1 = (x1 + ks1).astype(np.uint32)
    ks = [ks0, ks1, ks2]
    rots = ((13, 15, 26, 6), (17, 29, 16, 24))
    inject = [(1, 2), (2, 0), (0, 1), (1, 2), (2, 0)]
    for i in range(5):
        for r in rots[i % 2]:
            x0 = (x0 + x1).astype(np.uint32)
            x1 = rotl(x1, r)
            x1 = x1 ^ x0
        a, b = inject[i]
        x0 = (x0 + ks[a]).astype(np.uint32)
        x1 = (x1 + ks[b] + np.uint32(i + 1)).astype(np.uint32)
    return x0, x1


@functools.lru_cache(maxsize=4)
def _keep_mask_i32(nnz: int):
    i = np.arange(nnz, dtype=np.uint32)
    y0, y1 = _np_threefry2x32(0, 42, np.zeros(nnz, np.uint32), i)
    bits = y0 ^ y1
    u = ((bits >> np.uint32(9)) | np.uint32(0x3F800000)).view(np.float32)
    u = u - np.float32(1.0)
    drop = (u + np.float32(0.2)) > np.float32(1.0)
    return np.where(drop, np.uint32(0), np.uint32(0xFFFFFFFF)).view(np.int32)


def _select_block(v_ref, m_ref, i_ref, o_ref, oi_ref):
    vi = jax.lax.bitcast_convert_type(v_ref[...], jnp.int32)
    o_ref[...] = jax.lax.bitcast_convert_type(vi & m_ref[...], jnp.float32)
    oi_ref[...] = i_ref[...]


def kernel(adj_indices, adj_values):
    nnz = adj_values.shape[0]
    block = 524288
    grid = pl.cdiv(nnz, block)
    mask = jnp.asarray(_keep_mask_i32(nnz))
    val, idx = pl.pallas_call(
        _select_block,
        grid=(grid,),
        in_specs=[pl.BlockSpec((block,), lambda i: (i,)),
                  pl.BlockSpec((block,), lambda i: (i,)),
                  pl.BlockSpec((2, block), lambda i: (0, i))],
        out_specs=[pl.BlockSpec((block,), lambda i: (i,)),
                   pl.BlockSpec((2, block), lambda i: (0, i))],
        out_shape=[jax.ShapeDtypeStruct((nnz,), jnp.float32),
                   jax.ShapeDtypeStruct((2, nnz), adj_indices.dtype)],
    )(adj_values, mask, adj_indices)
    return (idx, val)


# speedup vs baseline: 20.6773x; 1.0800x over previous
import functools

import jax
import jax.numpy as jnp
import numpy as np
from jax.experimental import pallas as pl


def _np_threefry2x32(k0, k1, x0, x1):
    x0 = np.asarray(x0, np.uint32).copy()
    x1 = np.asarray(x1, np.uint32).copy()
    ks0 = np.uint32(k0)
    ks1 = np.uint32(k1)
    ks2 = np.uint32(ks0 ^ ks1 ^ np.uint32(0x1BD11BDA))

    def rotl(v, d):
        return ((v << np.uint32(d)) | (v >> np.uint32(32 - d))).astype(np.uint32)

    x0 = (x0 + ks0).astype(np.uint32)
    x1 = (x1 + ks1).astype(np.uint32)
    ks = [ks0, ks1, ks2]
    rots = ((13, 15, 26, 6), (17, 29, 16, 24))
    inject = [(1, 2), (2, 0), (0, 1), (1, 2), (2, 0)]
    for i in range(5):
        for r in rots[i % 2]:
            x0 = (x0 + x1).astype(np.uint32)
            x1 = rotl(x1, r)
            x1 = x1 ^ x0
        a, b = inject[i]
        x0 = (x0 + ks[a]).astype(np.uint32)
        x1 = (x1 + ks[b] + np.uint32(i + 1)).astype(np.uint32)
    return x0, x1


@functools.lru_cache(maxsize=4)
def _keep_mask_i32(nnz: int):
    i = np.arange(nnz, dtype=np.uint32)
    y0, y1 = _np_threefry2x32(0, 42, np.zeros(nnz, np.uint32), i)
    bits = y0 ^ y1
    u = ((bits >> np.uint32(9)) | np.uint32(0x3F800000)).view(np.float32)
    u = u - np.float32(1.0)
    drop = (u + np.float32(0.2)) > np.float32(1.0)
    return np.where(drop, np.uint32(0), np.uint32(0xFFFFFFFF)).view(np.int32)


def _select_block(v_ref, m_ref, i_ref, o_ref, oi_ref):
    vi = jax.lax.bitcast_convert_type(v_ref[...], jnp.int32)
    o_ref[...] = jax.lax.bitcast_convert_type(vi & m_ref[...], jnp.float32)
    oi_ref[...] = i_ref[...]


def kernel(adj_indices, adj_values):
    nnz = adj_values.shape[0]
    block = 1048576
    grid = pl.cdiv(nnz, block)
    mask = jnp.asarray(_keep_mask_i32(nnz))
    val, idx = pl.pallas_call(
        _select_block,
        grid=(grid,),
        in_specs=[pl.BlockSpec((block,), lambda i: (i,)),
                  pl.BlockSpec((block,), lambda i: (i,)),
                  pl.BlockSpec((2, block), lambda i: (0, i))],
        out_specs=[pl.BlockSpec((block,), lambda i: (i,)),
                   pl.BlockSpec((2, block), lambda i: (0, i))],
        out_shape=[jax.ShapeDtypeStruct((nnz,), jnp.float32),
                   jax.ShapeDtypeStruct((2, nnz), adj_indices.dtype)],
    )(adj_values, mask, adj_indices)
    return (idx, val)
